# Initial kernel scaffold; baseline (speedup 1.0000x reference)
import jax
import jax.numpy as jnp
from jax.experimental import pallas as pl


def kernel(word_ids, tag_ids, deprel_ids, word_emb, tag_emb, deprel_emb, W_w, W_t, W_d, b1, U, b2):
    raise NotImplementedError("write your pallas kernel here")


# speedup vs baseline: 2.9531x; 2.9531x over previous
import functools

import jax
import jax.numpy as jnp
from jax import lax
from jax.experimental import pallas as pl
from jax.experimental.pallas import tpu as pltpu
from jax.experimental.pallas import tpu_sc as plsc

B = 16384
NW, NT, ND = 18, 18, 12
E = 64
H = 256
C = 75

NC, NS = 2, 16
NWORKERS = NC * NS
CH = 128


def _make_sc_gather():
    n_w = B * NW // (NWORKERS * CH)
    n_t = B * NT // (NWORKERS * CH)
    n_d = B * ND // (NWORKERS * CH)
    mesh = plsc.VectorSubcoreMesh(core_axis_name="c", subcore_axis_name="s")

    @functools.partial(
        pl.kernel,
        out_type=[
            jax.ShapeDtypeStruct((B * NW, E), jnp.float32),
            jax.ShapeDtypeStruct((B * NT, E), jnp.float32),
            jax.ShapeDtypeStruct((B * ND, E), jnp.float32),
        ],
        mesh=mesh,
        compiler_params=pltpu.CompilerParams(use_tc_tiling_on_sc=False),
        scratch_types=[
            pltpu.VMEM((n_w, CH), jnp.int32),
            pltpu.VMEM((n_t, CH), jnp.int32),
            pltpu.VMEM((n_d, CH), jnp.int32),
            pltpu.VMEM((CH, E), jnp.float32),
            pltpu.SemaphoreType.DMA,
        ],
    )
    def sc_gather(wids, tids, dids, wemb, temb, demb, xw, xt, xd,
                  widx_v, tidx_v, didx_v, buf, gsem):
        wid = lax.axis_index("s") * NC + lax.axis_index("c")

        def run(idx_hbm, idx_v, table, out, n_chunks):
            pltpu.sync_copy(idx_hbm.at[pl.ds(wid * n_chunks, n_chunks)], idx_v)

            def body(c, carry):
                pltpu.async_copy(table.at[idx_v.at[c]], buf, gsem).wait()
                pltpu.sync_copy(
                    buf, out.at[pl.ds((wid * n_chunks + c) * CH, CH)])
                return carry

            lax.fori_loop(0, n_chunks, body, 0)

        run(wids, widx_v, wemb, xw, n_w)
        run(tids, tidx_v, temb, xt, n_t)
        run(dids, didx_v, demb, xd, n_d)

    return sc_gather


def _mlp(xw, xt, xd, W_w, W_t, W_d, b1, U, b2):
    bm = 256
    grid = (B // bm,)

    def body(xw_ref, xt_ref, xd_ref, ww_ref, wt_ref, wd_ref, b1_ref, u_ref,
             b2_ref, o_ref):
        z = jnp.dot(xw_ref[...], ww_ref[...], preferred_element_type=jnp.float32)
        z = z + jnp.dot(xt_ref[...], wt_ref[...], preferred_element_type=jnp.float32)
        z = z + jnp.dot(xd_ref[...], wd_ref[...], preferred_element_type=jnp.float32)
        z = z + b1_ref[...]
        h = jnp.maximum(z, 0.0)
        o_ref[...] = jnp.dot(h, u_ref[...], preferred_element_type=jnp.float32) + b2_ref[...]

    return pl.pallas_call(
        body,
        grid=grid,
        in_specs=[
            pl.BlockSpec((bm, NW * E), lambda i: (i, 0)),
            pl.BlockSpec((bm, NT * E), lambda i: (i, 0)),
            pl.BlockSpec((bm, ND * E), lambda i: (i, 0)),
            pl.BlockSpec((NW * E, H), lambda i: (0, 0)),
            pl.BlockSpec((NT * E, H), lambda i: (0, 0)),
            pl.BlockSpec((ND * E, H), lambda i: (0, 0)),
            pl.BlockSpec((1, H), lambda i: (0, 0)),
            pl.BlockSpec((H, C), lambda i: (0, 0)),
            pl.BlockSpec((1, C), lambda i: (0, 0)),
        ],
        out_specs=pl.BlockSpec((bm, C), lambda i: (i, 0)),
        out_shape=jax.ShapeDtypeStruct((B, C), jnp.float32),
    )(xw, xt, xd, W_w, W_t, W_d, b1.reshape(1, H), U, b2.reshape(1, C))


def kernel(word_ids, tag_ids, deprel_ids, word_emb, tag_emb, deprel_emb,
           W_w, W_t, W_d, b1, U, b2):
    wids = word_ids.reshape(-1, CH)
    tids = tag_ids.reshape(-1, CH)
    dids = deprel_ids.reshape(-1, CH)
    xw, xt, xd = _make_sc_gather()(wids, tids, dids,
                                   word_emb, tag_emb, deprel_emb)
    return _mlp(xw.reshape(B, NW * E), xt.reshape(B, NT * E),
                xd.reshape(B, ND * E), W_w, W_t, W_d, b1, U, b2)


# speedup vs baseline: 3.1503x; 1.0668x over previous
import functools

import jax
import jax.numpy as jnp
from jax import lax
from jax.experimental import pallas as pl
from jax.experimental.pallas import tpu as pltpu
from jax.experimental.pallas import tpu_sc as plsc

B = 16384
NW, NT, ND = 18, 18, 12
E = 64
H = 256
C = 75

NC, NS = 2, 16
NWORKERS = NC * NS
CH = 128


KB = 4
NSLOT = 3


def _make_sc_gather():
    n_w = B * NW // (NWORKERS * CH)
    n_t = B * NT // (NWORKERS * CH)
    n_d = B * ND // (NWORKERS * CH)
    mesh = plsc.VectorSubcoreMesh(core_axis_name="c", subcore_axis_name="s")

    @functools.partial(
        pl.kernel,
        out_type=[
            jax.ShapeDtypeStruct((B * NW, E), jnp.float32),
            jax.ShapeDtypeStruct((B * NT, E), jnp.float32),
            jax.ShapeDtypeStruct((B * ND, E), jnp.float32),
        ],
        mesh=mesh,
        compiler_params=pltpu.CompilerParams(use_tc_tiling_on_sc=False),
        scratch_types=[
            pltpu.VMEM((n_w * CH,), jnp.int32),
            pltpu.VMEM((n_t * CH,), jnp.int32),
            pltpu.VMEM((n_d * CH,), jnp.int32),
            pltpu.VMEM((NSLOT, KB * CH, E), jnp.float32),
            pltpu.SemaphoreType.DMA,
            pltpu.SemaphoreType.DMA,
        ],
    )
    def sc_gather(wids, tids, dids, wemb, temb, demb, xw, xt, xd,
                  widx_v, tidx_v, didx_v, buf, gsem, ssem):
        wid = lax.axis_index("s") * NC + lax.axis_index("c")

        def run(idx_hbm, idx_v, table, out, n_chunks):
            nblk = n_chunks // KB
            base = wid * n_chunks
            rows = KB * CH
            pltpu.sync_copy(idx_hbm.at[pl.ds(base * CH, n_chunks * CH)], idx_v)

            def g_pair(i):
                slot = lax.rem(i, NSLOT)
                return (table.at[idx_v.at[pl.ds(i * rows, rows)]],
                        buf.at[slot])

            def s_pair(i):
                slot = lax.rem(i, NSLOT)
                return (buf.at[slot],
                        out.at[pl.ds((base + i * KB) * CH, rows)])

            pltpu.async_copy(*g_pair(0), gsem)
            pltpu.async_copy(*g_pair(1), gsem)

            def body(i, carry):
                pltpu.make_async_copy(*g_pair(i), gsem).wait()
                pltpu.async_copy(*s_pair(i), ssem)

                @pl.when(i >= 1)
                def _():
                    pltpu.make_async_copy(*s_pair(i - 1), ssem).wait()

                @pl.when(i + 2 < nblk)
                def _():
                    pltpu.async_copy(*g_pair(i + 2), gsem)

                return carry

            lax.fori_loop(0, nblk, body, 0)
            pltpu.make_async_copy(*s_pair(nblk - 1), ssem).wait()

        run(wids, widx_v, wemb, xw, n_w)
        run(tids, tidx_v, temb, xt, n_t)
        run(dids, didx_v, demb, xd, n_d)

    return sc_gather


def _mlp(xw, xt, xd, W_w, W_t, W_d, b1, U, b2):
    bm = 256
    grid = (B // bm,)

    def body(xw_ref, xt_ref, xd_ref, ww_ref, wt_ref, wd_ref, b1_ref, u_ref,
             b2_ref, o_ref):
        z = jnp.dot(xw_ref[...], ww_ref[...], preferred_element_type=jnp.float32)
        z = z + jnp.dot(xt_ref[...], wt_ref[...], preferred_element_type=jnp.float32)
        z = z + jnp.dot(xd_ref[...], wd_ref[...], preferred_element_type=jnp.float32)
        z = z + b1_ref[...]
        h = jnp.maximum(z, 0.0)
        o_ref[...] = jnp.dot(h, u_ref[...], preferred_element_type=jnp.float32) + b2_ref[...]

    return pl.pallas_call(
        body,
        grid=grid,
        in_specs=[
            pl.BlockSpec((bm, NW * E), lambda i: (i, 0)),
            pl.BlockSpec((bm, NT * E), lambda i: (i, 0)),
            pl.BlockSpec((bm, ND * E), lambda i: (i, 0)),
            pl.BlockSpec((NW * E, H), lambda i: (0, 0)),
            pl.BlockSpec((NT * E, H), lambda i: (0, 0)),
            pl.BlockSpec((ND * E, H), lambda i: (0, 0)),
            pl.BlockSpec((1, H), lambda i: (0, 0)),
            pl.BlockSpec((H, C), lambda i: (0, 0)),
            pl.BlockSpec((1, C), lambda i: (0, 0)),
        ],
        out_specs=pl.BlockSpec((bm, C), lambda i: (i, 0)),
        out_shape=jax.ShapeDtypeStruct((B, C), jnp.float32),
    )(xw, xt, xd, W_w, W_t, W_d, b1.reshape(1, H), U, b2.reshape(1, C))


def kernel(word_ids, tag_ids, deprel_ids, word_emb, tag_emb, deprel_emb,
           W_w, W_t, W_d, b1, U, b2):
    wids = word_ids.reshape(-1)
    tids = tag_ids.reshape(-1)
    dids = deprel_ids.reshape(-1)
    xw, xt, xd = _make_sc_gather()(wids, tids, dids,
                                   word_emb, tag_emb, deprel_emb)
    return _mlp(xw.reshape(B, NW * E), xt.reshape(B, NT * E),
                xd.reshape(B, ND * E), W_w, W_t, W_d, b1, U, b2)
